# Initial kernel scaffold; baseline (speedup 1.0000x reference)
import jax
import jax.numpy as jnp
from jax.experimental import pallas as pl


def kernel(X, neighbors, W_self, W_neigh, bias):
    raise NotImplementedError("write your pallas kernel here")


# speedup vs baseline: 1.3535x; 1.3535x over previous
import functools

import jax
import jax.numpy as jnp
from jax import lax
from jax.experimental import pallas as pl
from jax.experimental.pallas import tpu as pltpu
from jax.experimental.pallas import tpu_sc as plsc

N = 10000
DEG = 32
IN_DIM = 128
OUT_DIM = 128

NC = 2
NS = 16
NW = NC * NS
B = 80
NBLK = 4
PER_W = B * NBLK
NPAD = NW * PER_W


def _sc_gather_sum(x, idx_all):
    mesh = plsc.VectorSubcoreMesh(core_axis_name="c", subcore_axis_name="s")

    @functools.partial(
        pl.kernel,
        out_type=jax.ShapeDtypeStruct((NPAD, IN_DIM), jnp.float32),
        mesh=mesh,
        scratch_types=[
            pltpu.VMEM((DEG, NBLK, B), jnp.int32),
            pltpu.VMEM((PER_W, IN_DIM), jnp.float32),
            pltpu.SemaphoreType.DMA,
        ],
    )
    def gather_sum(x_hbm, idx_hbm, out_hbm, idx_v, acc_v, sem):
        w = lax.axis_index("s") * NC + lax.axis_index("c")
        base = w * PER_W
        pltpu.sync_copy(idx_hbm.at[w], idx_v)

        descs = [
            pltpu.async_copy(
                x_hbm.at[idx_v.at[0, b]], acc_v.at[pl.ds(b * B, B)], sem
            )
            for b in range(NBLK)
        ]
        for d in descs:
            d.wait()

        @pl.loop(1, DEG)
        def _(k):
            ds = [
                pltpu.async_copy(
                    x_hbm.at[idx_v.at[k, b]],
                    acc_v.at[pl.ds(b * B, B)],
                    sem,
                    add=True,
                )
                for b in range(NBLK)
            ]
            for d in ds:
                d.wait()

        pltpu.sync_copy(acc_v, out_hbm.at[pl.ds(base, PER_W)])

    return gather_sum(x, idx_all)


def _tc_body(x_ref, s_ref, wst_ref, wnt_ref, b_ref, o_ref):
    h = jnp.dot(x_ref[...], wst_ref[...], preferred_element_type=jnp.float32)
    h = h + jnp.dot(s_ref[...], wnt_ref[...], preferred_element_type=jnp.float32)
    o_ref[...] = jnp.maximum(h + b_ref[...], 0.0)


def _tc_combine(x, neigh_sum, wst, wnt, bias2d):
    blk = 1000
    grid = N // blk
    return pl.pallas_call(
        _tc_body,
        grid=(grid,),
        in_specs=[
            pl.BlockSpec((blk, IN_DIM), lambda i: (i, 0)),
            pl.BlockSpec((blk, IN_DIM), lambda i: (i, 0)),
            pl.BlockSpec((IN_DIM, OUT_DIM), lambda i: (0, 0)),
            pl.BlockSpec((IN_DIM, OUT_DIM), lambda i: (0, 0)),
            pl.BlockSpec((1, OUT_DIM), lambda i: (0, 0)),
        ],
        out_specs=pl.BlockSpec((blk, OUT_DIM), lambda i: (i, 0)),
        out_shape=jax.ShapeDtypeStruct((N, OUT_DIM), jnp.float32),
    )(x, neigh_sum, wst, wnt, bias2d)


@jax.jit
def kernel(X, neighbors, W_self, W_neigh, bias):
    idx = neighbors.astype(jnp.int32)
    idx = jnp.pad(idx, ((0, NPAD - N), (0, 0)))
    idx_all = idx.reshape(NW, NBLK, B, DEG).transpose(0, 3, 1, 2)
    neigh_sum = _sc_gather_sum(X, idx_all)
    return _tc_combine(X, neigh_sum, W_self.T, W_neigh.T, bias.reshape(1, OUT_DIM))


# speedup vs baseline: 1.3584x; 1.0036x over previous
import functools

import jax
import jax.numpy as jnp
from jax import lax
from jax.experimental import pallas as pl
from jax.experimental.pallas import tpu as pltpu
from jax.experimental.pallas import tpu_sc as plsc

N = 10000
DEG = 32
IN_DIM = 128
OUT_DIM = 128

NC = 2
NS = 16
NW = NC * NS
CHUNK = 4
NCHUNK = 80
NBUF = 2
PER_W = CHUNK * NCHUNK
NPAD = NW * PER_W
VPR = IN_DIM // 16


def _sc_gather_sum(x, idx_all):
    mesh = plsc.VectorSubcoreMesh(core_axis_name="c", subcore_axis_name="s")

    @functools.partial(
        pl.kernel,
        out_type=jax.ShapeDtypeStruct((NPAD, IN_DIM), jnp.float32),
        mesh=mesh,
        scratch_types=[
            pltpu.VMEM((NCHUNK, CHUNK * DEG), jnp.int32),
            pltpu.VMEM((NBUF, CHUNK * DEG, IN_DIM), jnp.float32),
            pltpu.VMEM((PER_W, IN_DIM), jnp.float32),
            [pltpu.SemaphoreType.DMA] * NBUF,
        ],
    )
    def gather_sum(x_hbm, idx_hbm, out_hbm, idx_v, stage_v, res_v, sems):
        w = lax.axis_index("s") * NC + lax.axis_index("c")
        base = w * PER_W
        pltpu.sync_copy(idx_hbm.at[w], idx_v)

        def fire(c, b):
            pltpu.async_copy(x_hbm.at[idx_v.at[c]], stage_v.at[b], sems[b])

        def drain(c, b):
            pltpu.make_async_copy(
                x_hbm.at[idx_v.at[c]], stage_v.at[b], sems[b]
            ).wait()

        for b in range(NBUF):
            fire(b, b)

        @pl.loop(0, NCHUNK, step=NBUF)
        def _(c):
            for b in range(NBUF):
                cur = c + b
                drain(cur, b)
                stage = stage_v.at[b]
                for n in range(CHUNK):
                    zero = jnp.zeros((16,), jnp.float32)

                    @pl.loop(0, DEG, init_carry=(zero,) * VPR, unroll=4)
                    def acc(r, carry, stage=stage, n=n):
                        return tuple(
                            carry[v] + stage[n * DEG + r, pl.ds(v * 16, 16)]
                            for v in range(VPR)
                        )

                    for v in range(VPR):
                        res_v[cur * CHUNK + n, pl.ds(v * 16, 16)] = acc[v]

                nxt = cur + NBUF

                @pl.when(nxt < NCHUNK)
                def _(nxt=nxt, b=b):
                    fire(nxt, b)

        pltpu.sync_copy(res_v, out_hbm.at[pl.ds(base, PER_W)])

    return gather_sum(x, idx_all)


def _tc_body(x_ref, s_ref, wst_ref, wnt_ref, b_ref, o_ref):
    h = jnp.dot(x_ref[...], wst_ref[...], preferred_element_type=jnp.float32)
    h = h + jnp.dot(s_ref[...], wnt_ref[...], preferred_element_type=jnp.float32)
    o_ref[...] = jnp.maximum(h + b_ref[...], 0.0)


def _tc_combine(x, neigh_sum, wst, wnt, bias2d):
    blk = 1000
    grid = N // blk
    return pl.pallas_call(
        _tc_body,
        grid=(grid,),
        in_specs=[
            pl.BlockSpec((blk, IN_DIM), lambda i: (i, 0)),
            pl.BlockSpec((blk, IN_DIM), lambda i: (i, 0)),
            pl.BlockSpec((IN_DIM, OUT_DIM), lambda i: (0, 0)),
            pl.BlockSpec((IN_DIM, OUT_DIM), lambda i: (0, 0)),
            pl.BlockSpec((1, OUT_DIM), lambda i: (0, 0)),
        ],
        out_specs=pl.BlockSpec((blk, OUT_DIM), lambda i: (i, 0)),
        out_shape=jax.ShapeDtypeStruct((N, OUT_DIM), jnp.float32),
    )(x, neigh_sum, wst, wnt, bias2d)


@jax.jit
def kernel(X, neighbors, W_self, W_neigh, bias):
    idx = neighbors.astype(jnp.int32)
    idx = jnp.pad(idx, ((0, NPAD - N), (0, 0)))
    idx_all = idx.reshape(NW, NCHUNK, CHUNK * DEG)
    neigh_sum = _sc_gather_sum(X, idx_all)
    return _tc_combine(X, neigh_sum, W_self.T, W_neigh.T, bias.reshape(1, OUT_DIM))
